# baseline (speedup 1.0000x reference)
import dataclasses
import functools

import jax
import jax.numpy as jnp
from jax.experimental import pallas as pl
from jax.experimental.pallas import tpu as pltpu
from jax.experimental.pallas import tpu_sc as plsc

TOKENS = 16384
D_MODEL = 4096
N_EXPERTS = 64
TOP_K = 2
TB = 1024
LANES = 16
SC_WORKERS = 32


def _probs_kernel(x_ref, w_ref, probs_ref):
    x = x_ref[...]
    w = w_ref[...]
    logits = jax.lax.dot_general(
        x, w,
        dimension_numbers=(((1,), (1,)), ((), ())),
        preferred_element_type=jnp.float32,
        precision=jax.lax.Precision.DEFAULT,
    )
    m = jnp.max(logits, axis=-1, keepdims=True)
    e = jnp.exp(logits - m)
    probs_ref[...] = e / jnp.sum(e, axis=-1, keepdims=True)


def _tc_probs(x, w):
    grid = (TOKENS // TB,)
    return pl.pallas_call(
        _probs_kernel,
        grid=grid,
        in_specs=[
            pl.BlockSpec((TB, D_MODEL), lambda i: (i, 0)),
            pl.BlockSpec((N_EXPERTS, D_MODEL), lambda i: (0, 0)),
        ],
        out_specs=pl.BlockSpec((TB, N_EXPERTS), lambda i: (i, 0)),
        out_shape=jax.ShapeDtypeStruct((TOKENS, N_EXPERTS), jnp.float32),
        compiler_params=pltpu.CompilerParams(
            dimension_semantics=("parallel",),
        ),
    )(x, w)


def _sc_topk(probs):
    mesh = plsc.VectorSubcoreMesh(
        core_axis_name="core", subcore_axis_name="subcore")

    cp = pltpu.CompilerParams()
    if "needs_layout_passes" in pltpu.CompilerParams.__dataclass_fields__:
        cp = dataclasses.replace(cp, needs_layout_passes=False)

    nrows = TOKENS // LANES

    @pl.kernel(
        out_type=[
            jax.ShapeDtypeStruct((nrows, LANES), jnp.float32),
            jax.ShapeDtypeStruct((nrows, LANES), jnp.float32),
            jax.ShapeDtypeStruct((nrows, LANES), jnp.int32),
            jax.ShapeDtypeStruct((nrows, LANES), jnp.int32),
        ],
        mesh=mesh,
        compiler_params=cp,
    )
    def k(p_hbm, v1_hbm, v2_hbm, i1_hbm, i2_hbm):
        def body(p_vmem, v1_vmem, v2_vmem, i1_vmem, i2_vmem):
            lane = jax.lax.iota(jnp.int32, LANES)
            v1acc = jnp.zeros((LANES,), jnp.float32)
            v2acc = jnp.zeros((LANES,), jnp.float32)
            i1acc = jnp.zeros((LANES,), jnp.int32)
            i2acc = jnp.zeros((LANES,), jnp.int32)
            for t in range(LANES):
                vs = [p_vmem.at[t, pl.ds(LANES * j, LANES)][...]
                      for j in range(N_EXPERTS // LANES)]
                mm = vs[0]
                for v in vs[1:]:
                    mm = jnp.maximum(mm, v)
                m1 = jnp.max(mm)
                cand = None
                for j, v in enumerate(vs):
                    c = jnp.where(v == m1, lane + LANES * j, N_EXPERTS)
                    cand = c if cand is None else jnp.minimum(cand, c)
                i1 = jnp.min(cand)
                vs2 = [jnp.where(lane + LANES * j == i1, -1.0, v)
                       for j, v in enumerate(vs)]
                mm2 = vs2[0]
                for v in vs2[1:]:
                    mm2 = jnp.maximum(mm2, v)
                m2 = jnp.max(mm2)
                cand2 = None
                for j, v in enumerate(vs2):
                    c = jnp.where(v == m2, lane + LANES * j, N_EXPERTS)
                    cand2 = c if cand2 is None else jnp.minimum(cand2, c)
                i2 = jnp.min(cand2)
                sel = lane == t
                v1acc = jnp.where(sel, m1, v1acc)
                v2acc = jnp.where(sel, m2, v2acc)
                i1acc = jnp.where(sel, i1, i1acc)
                i2acc = jnp.where(sel, i2, i2acc)
            v1_vmem.at[0][...] = v1acc
            v2_vmem.at[0][...] = v2acc
            i1_vmem.at[0][...] = i1acc
            i2_vmem.at[0][...] = i2acc

        out_spec = pl.BlockSpec((1, LANES), lambda i: (i, 0))
        pltpu.emit_pipeline(
            body,
            grid=(TOKENS // LANES,),
            in_specs=[pl.BlockSpec((LANES, N_EXPERTS), lambda i: (i, 0))],
            out_specs=[out_spec, out_spec, out_spec, out_spec],
            core_axis_name=("core", "subcore"),
            dimension_semantics=(pltpu.PARALLEL,),
        )(p_hbm, v1_hbm, v2_hbm, i1_hbm, i2_hbm)

    return k(probs)


@functools.partial(jax.jit, static_argnames=())
def _gate(x, w):
    probs = _tc_probs(x, w)
    v1, v2, i1, i2 = _sc_topk(probs)
    vals = jnp.stack([v1.reshape(-1), v2.reshape(-1)], axis=1)
    idx = jnp.stack([i1.reshape(-1), i2.reshape(-1)], axis=1)
    return vals, idx


def kernel(input, wg_weight):
    vals, idx = _gate(input, wg_weight)
    aux_loss = jnp.array(0.0, dtype=jnp.float32)
    return (aux_loss, vals, idx, jnp.zeros((N_EXPERTS,), dtype=jnp.float32))


# speedup vs baseline: 1.5396x; 1.5396x over previous
import functools

import jax
import jax.numpy as jnp
from jax.experimental import pallas as pl
from jax.experimental.pallas import tpu as pltpu

TOKENS = 16384
D_MODEL = 4096
N_EXPERTS = 64
TOP_K = 2
TB = 1024
NSPLIT = 2
TBS = TB // NSPLIT


def _gate_kernel(xa_ref, xb_ref, w_ref, vals_ref, idx_ref, z_ref):
    w = w_ref[...]
    dots = []
    for x_ref in (xa_ref, xb_ref):
        dots.append(jax.lax.dot_general(
            x_ref[...], w,
            dimension_numbers=(((1,), (1,)), ((), ())),
            preferred_element_type=jnp.float32,
            precision=jax.lax.Precision.DEFAULT,
        ))
    logits = jnp.concatenate(dots, axis=0)
    m = jnp.max(logits, axis=-1, keepdims=True)
    e = jnp.exp(logits - m)
    probs = e / jnp.sum(e, axis=-1, keepdims=True)

    iota = jax.lax.broadcasted_iota(jnp.int32, probs.shape, 1)
    v1 = jnp.max(probs, axis=-1, keepdims=True)
    i1 = jnp.min(jnp.where(probs == v1, iota, N_EXPERTS), axis=-1, keepdims=True)
    masked = jnp.where(iota == i1, -jnp.inf, probs)
    v2 = jnp.max(masked, axis=-1, keepdims=True)
    i2 = jnp.min(jnp.where(masked == v2, iota, N_EXPERTS), axis=-1, keepdims=True)

    vals_ref[...] = jnp.concatenate([v1, v2], axis=1).T
    idx_ref[...] = jnp.concatenate([i1, i2], axis=1).T
    z_ref[...] = jnp.zeros_like(z_ref)


@functools.partial(jax.jit, static_argnames=())
def _gate(x, w):
    grid = (TOKENS // TB,)
    vals, idx, z = pl.pallas_call(
        _gate_kernel,
        grid=grid,
        in_specs=[
            pl.BlockSpec((TBS, D_MODEL), lambda i: (2 * i, 0)),
            pl.BlockSpec((TBS, D_MODEL), lambda i: (2 * i + 1, 0)),
            pl.BlockSpec((N_EXPERTS, D_MODEL), lambda i: (0, 0)),
        ],
        out_specs=[
            pl.BlockSpec((TOP_K, TB), lambda i: (0, i)),
            pl.BlockSpec((TOP_K, TB), lambda i: (0, i)),
            pl.BlockSpec((N_EXPERTS,), lambda i: (0,)),
        ],
        out_shape=[
            jax.ShapeDtypeStruct((TOP_K, TOKENS), jnp.float32),
            jax.ShapeDtypeStruct((TOP_K, TOKENS), jnp.int32),
            jax.ShapeDtypeStruct((N_EXPERTS,), jnp.float32),
        ],
        compiler_params=pltpu.CompilerParams(
            dimension_semantics=("parallel",),
        ),
    )(x, x, w)
    return vals.T, idx.T, z


def kernel(input, wg_weight):
    vals, idx, z = _gate(input, wg_weight)
    aux_loss = jnp.array(0.0, dtype=jnp.float32)
    return (aux_loss, vals, idx, z)


# speedup vs baseline: 1.5518x; 1.0079x over previous
import functools

import jax
import jax.numpy as jnp
from jax.experimental import pallas as pl
from jax.experimental.pallas import tpu as pltpu

TOKENS = 16384
D_MODEL = 4096
N_EXPERTS = 64
TOP_K = 2
TB = 1024
NSPLIT = 4
TBS = TB // NSPLIT


def _gate_kernel(xa_ref, xb_ref, xc_ref, xd_ref, w_ref, vals_ref, idx_ref, z_ref):
    w = w_ref[...]
    dots = []
    for x_ref in (xa_ref, xb_ref, xc_ref, xd_ref):
        dots.append(jax.lax.dot_general(
            x_ref[...], w,
            dimension_numbers=(((1,), (1,)), ((), ())),
            preferred_element_type=jnp.float32,
            precision=jax.lax.Precision.DEFAULT,
        ))
    logits = jnp.concatenate(dots, axis=0)
    m = jnp.max(logits, axis=-1, keepdims=True)
    e = jnp.exp(logits - m)
    probs = e / jnp.sum(e, axis=-1, keepdims=True)

    iota = jax.lax.broadcasted_iota(jnp.int32, probs.shape, 1)
    v1 = jnp.max(probs, axis=-1, keepdims=True)
    i1 = jnp.min(jnp.where(probs == v1, iota, N_EXPERTS), axis=-1, keepdims=True)
    masked = jnp.where(iota == i1, -jnp.inf, probs)
    v2 = jnp.max(masked, axis=-1, keepdims=True)
    i2 = jnp.min(jnp.where(masked == v2, iota, N_EXPERTS), axis=-1, keepdims=True)

    vals_ref[...] = jnp.concatenate([v1, v2], axis=1).T
    idx_ref[...] = jnp.concatenate([i1, i2], axis=1).T
    z_ref[...] = jnp.zeros_like(z_ref)


@functools.partial(jax.jit, static_argnames=())
def _gate(x, w):
    grid = (TOKENS // TB,)
    vals, idx, z = pl.pallas_call(
        _gate_kernel,
        grid=grid,
        in_specs=[
            pl.BlockSpec((TBS, D_MODEL), lambda i: (4 * i, 0)),
            pl.BlockSpec((TBS, D_MODEL), lambda i: (4 * i + 1, 0)),
            pl.BlockSpec((TBS, D_MODEL), lambda i: (4 * i + 2, 0)),
            pl.BlockSpec((TBS, D_MODEL), lambda i: (4 * i + 3, 0)),
            pl.BlockSpec((N_EXPERTS, D_MODEL), lambda i: (0, 0)),
        ],
        out_specs=[
            pl.BlockSpec((TOP_K, TB), lambda i: (0, i)),
            pl.BlockSpec((TOP_K, TB), lambda i: (0, i)),
            pl.BlockSpec((N_EXPERTS,), lambda i: (0,)),
        ],
        out_shape=[
            jax.ShapeDtypeStruct((TOP_K, TOKENS), jnp.float32),
            jax.ShapeDtypeStruct((TOP_K, TOKENS), jnp.int32),
            jax.ShapeDtypeStruct((N_EXPERTS,), jnp.float32),
        ],
        compiler_params=pltpu.CompilerParams(
            dimension_semantics=("parallel",),
        ),
    )(x, x, x, x, w)
    return vals.T, idx.T, z


def kernel(input, wg_weight):
    vals, idx, z = _gate(input, wg_weight)
    aux_loss = jnp.array(0.0, dtype=jnp.float32)
    return (aux_loss, vals, idx, z)
